# Initial kernel scaffold; baseline (speedup 1.0000x reference)
import jax
import jax.numpy as jnp
from jax.experimental import pallas as pl


def kernel(points, spoints):
    raise NotImplementedError("write your pallas kernel here")


# speedup vs baseline: 5.1250x; 5.1250x over previous
import jax
import jax.numpy as jnp
from jax.experimental import pallas as pl
from jax.experimental.pallas import tpu as pltpu

_K = 11
_QB = 128


def _voro_kernel(p_ref, s_ref, out_ref):
    p = p_ref[...]
    s = s_ref[0]
    qb = p.shape[0]
    m = s.shape[1]

    ps = jax.lax.dot_general(
        p.astype(jnp.bfloat16), s.astype(jnp.bfloat16),
        (((1,), (0,)), ((), ())),
        preferred_element_type=jnp.float32,
    )
    p2 = jnp.sum(p * p, axis=1, keepdims=True)
    s2 = s[3:4, :]
    d2 = (p2 + s2) - 2.0 * ps

    iota = jax.lax.broadcasted_iota(jnp.int32, (qb, m), 1)
    big = jnp.int32(m)
    inf = jnp.float32(jnp.inf)

    lane = jax.lax.broadcasted_iota(jnp.int32, (qb, 8), 1)
    lmask = (lane < 3).astype(jnp.float32)

    c0 = None
    ptv = None
    best = jnp.full((qb, 1), inf, dtype=jnp.float32)
    for k in range(_K):
        mn = jnp.min(d2, axis=1, keepdims=True)
        idx = jnp.min(jnp.where(d2 == mn, iota, big), axis=1, keepdims=True)
        mask = iota == idx
        row = jax.lax.dot_general(
            mask.astype(jnp.float32), s, (((1,), (1,)), ((), ())),
            preferred_element_type=jnp.float32,
            precision=jax.lax.Precision.HIGHEST,
        )
        if k == 0:
            c0 = row * lmask
            ptv = p * lmask - c0
        else:
            cj = row * lmask
            edge = cj - c0
            e2 = jnp.sum(edge * edge, axis=1, keepdims=True)
            el = jnp.sqrt(e2)
            vl = jnp.sum(ptv * edge, axis=1, keepdims=True) / el
            sq = (vl - el * 0.5) ** 2
            best = jnp.minimum(best, sq)
        if k + 1 < _K:
            d2 = jnp.where(mask, inf, d2)

    out_ref[...] = best


def kernel(points, spoints):
    B, N, _ = points.shape
    M = spoints.shape[1]

    zeros = jnp.zeros((B, N, 5), dtype=jnp.float32)
    p_pad = jnp.concatenate([points, zeros], axis=-1).reshape(B * N, 8)

    st = jnp.swapaxes(spoints, 1, 2)
    s2 = jnp.sum(spoints * spoints, axis=-1)[:, None, :]
    sz = jnp.zeros((B, 4, M), dtype=jnp.float32)
    s_aug = jnp.concatenate([st, s2, sz], axis=1)

    nblk = N // _QB
    grid = (B * nblk,)

    out = pl.pallas_call(
        _voro_kernel,
        grid=grid,
        in_specs=[
            pl.BlockSpec((_QB, 8), lambda i: (i, 0)),
            pl.BlockSpec((1, 8, M), lambda i, _n=nblk: (i // _n, 0, 0)),
        ],
        out_specs=pl.BlockSpec((_QB, 1), lambda i: (i, 0)),
        out_shape=jax.ShapeDtypeStruct((B * N, 1), jnp.float32),
    )(p_pad, s_aug)

    return out.reshape(B, N)


# speedup vs baseline: 5.1650x; 1.0078x over previous
import jax
import jax.numpy as jnp
from jax.experimental import pallas as pl
from jax.experimental.pallas import tpu as pltpu

_K = 11
_QB = 128


def _voro_kernel(p_ref, s_ref, out_ref):
    p = p_ref[...]
    s = s_ref[0]
    qb = p.shape[0]
    m = s.shape[1]

    ps = jax.lax.dot_general(
        p.astype(jnp.bfloat16), s.astype(jnp.bfloat16),
        (((1,), (0,)), ((), ())),
        preferred_element_type=jnp.float32,
    )
    p2 = jnp.sum(p * p, axis=1, keepdims=True)
    s2 = s[3:4, :]
    d2 = (p2 + s2) - 2.0 * ps

    iota = jax.lax.broadcasted_iota(jnp.int32, (qb, m), 1)
    big = jnp.float32(3.0e38)
    inf = jnp.float32(jnp.inf)

    lane = jax.lax.broadcasted_iota(jnp.int32, (qb, 8), 1)
    lmask = (lane < 3).astype(jnp.float32)

    c0 = None
    ptv = None
    best = jnp.full((qb, 1), inf, dtype=jnp.float32)
    for k in range(_K):
        idx = jnp.argmin(d2, axis=1).reshape(qb, 1)
        maskf = (iota == idx).astype(jnp.float32)
        row = jax.lax.dot_general(
            maskf, s, (((1,), (1,)), ((), ())),
            preferred_element_type=jnp.float32,
            precision=jax.lax.Precision.HIGHEST,
        )
        if k == 0:
            c0 = row * lmask
            ptv = p * lmask - c0
        else:
            cj = row * lmask
            edge = cj - c0
            e2 = jnp.sum(edge * edge, axis=1, keepdims=True)
            el = jnp.sqrt(e2)
            vl = jnp.sum(ptv * edge, axis=1, keepdims=True) / el
            sq = (vl - el * 0.5) ** 2
            best = jnp.minimum(best, sq)
        if k + 1 < _K:
            d2 = d2 + maskf * big

    out_ref[...] = best


def kernel(points, spoints):
    B, N, _ = points.shape
    M = spoints.shape[1]

    zeros = jnp.zeros((B, N, 5), dtype=jnp.float32)
    p_pad = jnp.concatenate([points, zeros], axis=-1).reshape(B * N, 8)

    st = jnp.swapaxes(spoints, 1, 2)
    s2 = jnp.sum(spoints * spoints, axis=-1)[:, None, :]
    sz = jnp.zeros((B, 4, M), dtype=jnp.float32)
    s_aug = jnp.concatenate([st, s2, sz], axis=1)

    nblk = N // _QB
    grid = (B * nblk,)

    out = pl.pallas_call(
        _voro_kernel,
        grid=grid,
        in_specs=[
            pl.BlockSpec((_QB, 8), lambda i: (i, 0)),
            pl.BlockSpec((1, 8, M), lambda i, _n=nblk: (i // _n, 0, 0)),
        ],
        out_specs=pl.BlockSpec((_QB, 1), lambda i: (i, 0)),
        out_shape=jax.ShapeDtypeStruct((B * N, 1), jnp.float32),
    )(p_pad, s_aug)

    return out.reshape(B, N)


# speedup vs baseline: 11.2541x; 2.1789x over previous
import jax
import jax.numpy as jnp
from jax.experimental import pallas as pl
from jax.experimental.pallas import tpu as pltpu

_K = 11
_QB = 128


def _voro_kernel(p_ref, s_ref, sr_ref, out_ref):
    p = p_ref[...]
    s = s_ref[0]
    sr = sr_ref[0]
    qb = p.shape[0]
    m = s.shape[1]

    ps = jax.lax.dot_general(
        p.astype(jnp.bfloat16), s.astype(jnp.bfloat16),
        (((1,), (0,)), ((), ())),
        preferred_element_type=jnp.float32,
    )
    p2 = jnp.sum(p * p, axis=1, keepdims=True)
    s2 = s[3:4, :]
    d2 = (p2 + s2) - 2.0 * ps

    iota = jax.lax.broadcasted_iota(jnp.int32, (qb, m), 1)
    big = jnp.float32(3.0e38)
    inf = jnp.float32(jnp.inf)

    iota128 = jax.lax.broadcasted_iota(jnp.int32, (qb, 128), 1)
    iota512 = jax.lax.broadcasted_iota(jnp.int32, (qb, 512), 1)
    h512 = jnp.bitwise_and(iota512, 63)

    lo_ohs = []
    his = []
    for k in range(_K):
        idx = jnp.argmin(d2, axis=1).reshape(qb, 1)
        his.append(jnp.right_shift(idx, 7))
        lo = jnp.bitwise_and(idx, 127)
        lo_ohs.append((iota128 == lo).astype(jnp.float32))
        if k + 1 < _K:
            maskf = (iota == idx).astype(jnp.float32)
            d2 = d2 + maskf * big

    lo_stack = jnp.concatenate(lo_ohs, axis=0)
    t_all = jax.lax.dot_general(
        lo_stack, sr, (((1,), (0,)), ((), ())),
        preferred_element_type=jnp.float32,
        precision=jax.lax.Precision.HIGHEST,
    )

    lane = jax.lax.broadcasted_iota(jnp.int32, (qb, 8), 1)
    lmask = (lane < 3).astype(jnp.float32)

    rows = []
    for k in range(_K):
        t_k = t_all[k * qb:(k + 1) * qb, :]
        m64 = (h512 == his[k]).astype(jnp.float32)
        psel = t_k * m64
        rows.append(jnp.sum(psel.reshape(qb, 8, 64), axis=2))

    c0 = rows[0] * lmask
    ptv = p * lmask - c0
    best = jnp.full((qb, 1), inf, dtype=jnp.float32)
    for j in range(1, _K):
        cj = rows[j] * lmask
        edge = cj - c0
        e2 = jnp.sum(edge * edge, axis=1, keepdims=True)
        el = jnp.sqrt(e2)
        vl = jnp.sum(ptv * edge, axis=1, keepdims=True) / el
        sq = (vl - el * 0.5) ** 2
        best = jnp.minimum(best, sq)

    out_ref[...] = best


def kernel(points, spoints):
    B, N, _ = points.shape
    M = spoints.shape[1]

    zeros = jnp.zeros((B, N, 5), dtype=jnp.float32)
    p_pad = jnp.concatenate([points, zeros], axis=-1).reshape(B * N, 8)

    st = jnp.swapaxes(spoints, 1, 2)
    s2 = jnp.sum(spoints * spoints, axis=-1)[:, None, :]
    sz = jnp.zeros((B, 4, M), dtype=jnp.float32)
    s_aug = jnp.concatenate([st, s2, sz], axis=1)

    sr = s_aug.reshape(B, 8, 64, 128).transpose(0, 3, 1, 2).reshape(B, 128, 512)

    nblk = N // _QB
    grid = (B * nblk,)

    out = pl.pallas_call(
        _voro_kernel,
        grid=grid,
        in_specs=[
            pl.BlockSpec((_QB, 8), lambda i: (i, 0)),
            pl.BlockSpec((1, 8, M), lambda i, _n=nblk: (i // _n, 0, 0)),
            pl.BlockSpec((1, 128, 512), lambda i, _n=nblk: (i // _n, 0, 0)),
        ],
        out_specs=pl.BlockSpec((_QB, 1), lambda i: (i, 0)),
        out_shape=jax.ShapeDtypeStruct((B * N, 1), jnp.float32),
    )(p_pad, s_aug, sr)

    return out.reshape(B, N)


# speedup vs baseline: 11.2810x; 1.0024x over previous
import jax
import jax.numpy as jnp
from jax.experimental import pallas as pl
from jax.experimental.pallas import tpu as pltpu

_K = 11
_QB = 256


def _voro_kernel(p_ref, s_ref, sr_ref, out_ref):
    p = p_ref[...]
    s = s_ref[0]
    sr = sr_ref[0]
    qb = p.shape[0]
    m = s.shape[1]

    ps = jax.lax.dot_general(
        p.astype(jnp.bfloat16), s.astype(jnp.bfloat16),
        (((1,), (0,)), ((), ())),
        preferred_element_type=jnp.float32,
    )
    p2 = jnp.sum(p * p, axis=1, keepdims=True)
    s2 = s[3:4, :]
    d2 = (p2 + s2) - 2.0 * ps

    iota = jax.lax.broadcasted_iota(jnp.int32, (qb, m), 1)
    big = jnp.float32(3.0e38)
    inf = jnp.float32(jnp.inf)

    iota128 = jax.lax.broadcasted_iota(jnp.int32, (qb, 128), 1)
    iota512 = jax.lax.broadcasted_iota(jnp.int32, (qb, 512), 1)
    h512 = jnp.bitwise_and(iota512, 63)

    lo_ohs = []
    his = []
    for k in range(_K):
        idx = jnp.argmin(d2, axis=1).reshape(qb, 1)
        his.append(jnp.right_shift(idx, 7))
        lo = jnp.bitwise_and(idx, 127)
        lo_ohs.append((iota128 == lo).astype(jnp.float32))
        if k + 1 < _K:
            d2 = jnp.where(iota == idx, big, d2)

    lo_stack = jnp.concatenate(lo_ohs, axis=0)
    t_all = jax.lax.dot_general(
        lo_stack, sr, (((1,), (0,)), ((), ())),
        preferred_element_type=jnp.float32,
        precision=jax.lax.Precision.HIGHEST,
    )

    lane = jax.lax.broadcasted_iota(jnp.int32, (qb, 8), 1)
    lmask = (lane < 3).astype(jnp.float32)

    rows = []
    for k in range(_K):
        t_k = t_all[k * qb:(k + 1) * qb, :]
        m64 = (h512 == his[k]).astype(jnp.float32)
        psel = t_k * m64
        rows.append(jnp.sum(psel.reshape(qb, 8, 64), axis=2))

    c0 = rows[0] * lmask
    ptv = p * lmask - c0
    best = jnp.full((qb, 1), inf, dtype=jnp.float32)
    for j in range(1, _K):
        cj = rows[j] * lmask
        edge = cj - c0
        e2 = jnp.sum(edge * edge, axis=1, keepdims=True)
        el = jnp.sqrt(e2)
        vl = jnp.sum(ptv * edge, axis=1, keepdims=True) / el
        sq = (vl - el * 0.5) ** 2
        best = jnp.minimum(best, sq)

    out_ref[...] = best


def kernel(points, spoints):
    B, N, _ = points.shape
    M = spoints.shape[1]

    zeros = jnp.zeros((B, N, 5), dtype=jnp.float32)
    p_pad = jnp.concatenate([points, zeros], axis=-1).reshape(B * N, 8)

    st = jnp.swapaxes(spoints, 1, 2)
    s2 = jnp.sum(spoints * spoints, axis=-1)[:, None, :]
    sz = jnp.zeros((B, 4, M), dtype=jnp.float32)
    s_aug = jnp.concatenate([st, s2, sz], axis=1)

    sr = s_aug.reshape(B, 8, 64, 128).transpose(0, 3, 1, 2).reshape(B, 128, 512)

    nblk = N // _QB
    grid = (B * nblk,)

    out = pl.pallas_call(
        _voro_kernel,
        grid=grid,
        in_specs=[
            pl.BlockSpec((_QB, 8), lambda i: (i, 0)),
            pl.BlockSpec((1, 8, M), lambda i, _n=nblk: (i // _n, 0, 0)),
            pl.BlockSpec((1, 128, 512), lambda i, _n=nblk: (i // _n, 0, 0)),
        ],
        out_specs=pl.BlockSpec((_QB, 1), lambda i: (i, 0)),
        out_shape=jax.ShapeDtypeStruct((B * N, 1), jnp.float32),
        compiler_params=pltpu.CompilerParams(
            dimension_semantics=("parallel",)),
    )(p_pad, s_aug, sr)

    return out.reshape(B, N)


# speedup vs baseline: 16.8593x; 1.4945x over previous
import jax
import jax.numpy as jnp
from jax.experimental import pallas as pl
from jax.experimental.pallas import tpu as pltpu

_K = 11
_QB = 256


def _voro_kernel(p_ref, s_ref, sr_ref, out_ref):
    p = p_ref[...]
    s = s_ref[0]
    sr = sr_ref[0]
    qb = p.shape[0]
    m = s.shape[1]

    ps = jax.lax.dot_general(
        p.astype(jnp.bfloat16), s.astype(jnp.bfloat16),
        (((1,), (0,)), ((), ())),
        preferred_element_type=jnp.float32,
    )
    p2 = jnp.sum(p * p, axis=1, keepdims=True)
    s2 = s[3:4, :]
    d2 = (p2 + s2) - 2.0 * ps

    iota = jax.lax.broadcasted_iota(jnp.int32, (qb, m), 1)
    big = jnp.float32(3.0e38)
    bigi = jnp.int32(m)
    inf = jnp.float32(jnp.inf)

    iota128 = jax.lax.broadcasted_iota(jnp.int32, (qb, 128), 1)
    iota512 = jax.lax.broadcasted_iota(jnp.int32, (_K * qb, 512), 1)
    h512 = jnp.bitwise_and(iota512, 63)

    lo_ohs = []
    his = []
    for k in range(_K):
        idx = jnp.argmin(d2, axis=1).reshape(qb, 1)
        his.append(jnp.right_shift(idx, 7))
        lo = jnp.bitwise_and(idx, 127)
        lo_ohs.append((iota128 == lo).astype(jnp.float32))
        if k + 1 < _K:
            d2 = jnp.where(iota == idx, big, d2)

    lo_stack = jnp.concatenate(lo_ohs, axis=0)
    t_all = jax.lax.dot_general(
        lo_stack, sr, (((1,), (0,)), ((), ())),
        preferred_element_type=jnp.float32,
        precision=jax.lax.Precision.HIGHEST,
    )

    lane = jax.lax.broadcasted_iota(jnp.int32, (qb, 8), 1)
    lmask = (lane < 3).astype(jnp.float32)

    his_stack = jnp.concatenate(his, axis=0)
    m64s = (h512 == his_stack).astype(jnp.float32)
    psel = t_all * m64s
    omap = (jnp.right_shift(
        jax.lax.broadcasted_iota(jnp.int32, (512, 8), 0), 6)
        == jax.lax.broadcasted_iota(jnp.int32, (512, 8), 1)
    ).astype(jnp.float32)
    rows_all = jax.lax.dot_general(
        psel, omap, (((1,), (0,)), ((), ())),
        preferred_element_type=jnp.float32,
        precision=jax.lax.Precision.HIGHEST,
    )
    rows = [rows_all[k * qb:(k + 1) * qb, :] for k in range(_K)]

    c0 = rows[0] * lmask
    ptv = p * lmask - c0
    best = jnp.full((qb, 1), inf, dtype=jnp.float32)
    for j in range(1, _K):
        cj = rows[j] * lmask
        edge = cj - c0
        e2 = jnp.sum(edge * edge, axis=1, keepdims=True)
        el = jnp.sqrt(e2)
        vl = jnp.sum(ptv * edge, axis=1, keepdims=True) / el
        sq = (vl - el * 0.5) ** 2
        best = jnp.minimum(best, sq)

    out_ref[...] = best


def kernel(points, spoints):
    B, N, _ = points.shape
    M = spoints.shape[1]

    zeros = jnp.zeros((B, N, 5), dtype=jnp.float32)
    p_pad = jnp.concatenate([points, zeros], axis=-1).reshape(B * N, 8)

    st = jnp.swapaxes(spoints, 1, 2)
    s2 = jnp.sum(spoints * spoints, axis=-1)[:, None, :]
    sz = jnp.zeros((B, 4, M), dtype=jnp.float32)
    s_aug = jnp.concatenate([st, s2, sz], axis=1)

    sr = s_aug.reshape(B, 8, 64, 128).transpose(0, 3, 1, 2).reshape(B, 128, 512)

    nblk = N // _QB
    grid = (B * nblk,)

    out = pl.pallas_call(
        _voro_kernel,
        grid=grid,
        in_specs=[
            pl.BlockSpec((_QB, 8), lambda i: (i, 0)),
            pl.BlockSpec((1, 8, M), lambda i, _n=nblk: (i // _n, 0, 0)),
            pl.BlockSpec((1, 128, 512), lambda i, _n=nblk: (i // _n, 0, 0)),
        ],
        out_specs=pl.BlockSpec((_QB, 1), lambda i: (i, 0)),
        out_shape=jax.ShapeDtypeStruct((B * N, 1), jnp.float32),
        compiler_params=pltpu.CompilerParams(
            dimension_semantics=("parallel",)),
    )(p_pad, s_aug, sr)

    return out.reshape(B, N)
